# baseline (speedup 1.0000x reference)
import jax
import jax.numpy as jnp
from jax.experimental import pallas as pl


def _add_body(x_ref, pe_ref, o_ref):
    o_ref[...] = x_ref[...] + pe_ref[...]


def kernel(x, pos_embed):
    S, B, D = x.shape
    pe3 = pos_embed.reshape(S, 1, D)
    grid = (S,)
    return pl.pallas_call(
        _add_body,
        grid=grid,
        in_specs=[
            pl.BlockSpec((1, B, D), lambda s: (s, 0, 0)),
            pl.BlockSpec((1, 1, D), lambda s: (s, 0, 0)),
        ],
        out_specs=pl.BlockSpec((1, B, D), lambda s: (s, 0, 0)),
        out_shape=jax.ShapeDtypeStruct((S, B, D), x.dtype),
    )(x, pe3)


# speedup vs baseline: 2.1688x; 2.1688x over previous
import jax
import jax.numpy as jnp
from jax.experimental import pallas as pl


def _add_body(x_ref, pe_ref, o_ref):
    o_ref[...] = x_ref[...] + pe_ref[...]


def kernel(x, pos_embed):
    S, B, D = x.shape
    pe3 = pos_embed.reshape(S, 1, D)
    BS = 10
    grid = (S // BS,)
    return pl.pallas_call(
        _add_body,
        grid=grid,
        in_specs=[
            pl.BlockSpec((BS, B, D), lambda s: (s, 0, 0)),
            pl.BlockSpec((BS, 1, D), lambda s: (s, 0, 0)),
        ],
        out_specs=pl.BlockSpec((BS, B, D), lambda s: (s, 0, 0)),
        out_shape=jax.ShapeDtypeStruct((S, B, D), x.dtype),
    )(x, pe3)


# speedup vs baseline: 2.1804x; 1.0053x over previous
import jax
import jax.numpy as jnp
from jax.experimental import pallas as pl


def _add_body(x_ref, pe_ref, o_ref):
    o_ref[...] = x_ref[...] + pe_ref[...]


def kernel(x, pos_embed):
    S, B, D = x.shape
    pe3 = pos_embed.reshape(S, 1, D)
    BS = 25
    grid = (S // BS,)
    return pl.pallas_call(
        _add_body,
        grid=grid,
        in_specs=[
            pl.BlockSpec((BS, B, D), lambda s: (s, 0, 0)),
            pl.BlockSpec((BS, 1, D), lambda s: (s, 0, 0)),
        ],
        out_specs=pl.BlockSpec((BS, B, D), lambda s: (s, 0, 0)),
        out_shape=jax.ShapeDtypeStruct((S, B, D), x.dtype),
    )(x, pe3)
